# Initial kernel scaffold; baseline (speedup 1.0000x reference)
import jax
import jax.numpy as jnp
from jax.experimental import pallas as pl


def kernel(feat_map0, feat_map1, feat_map2, W, b):
    raise NotImplementedError("write your pallas kernel here")


# speedup vs baseline: 14.2900x; 14.2900x over previous
import jax
import jax.numpy as jnp
from jax.experimental import pallas as pl

_CHUNK = 2048


def _linear_kernel(x_ref, w_ref, b_ref, o_ref):
    x = x_ref[0]
    o_ref[0] = jnp.dot(w_ref[...], x, preferred_element_type=jnp.float32) + b_ref[...]


def kernel(feat_map0, feat_map1, feat_map2, W, b):
    B, C, H, Wd = feat_map0.shape
    P = H * Wd
    x = feat_map0.reshape(B, C, P)
    b2 = b.reshape(C, 1)
    out = pl.pallas_call(
        _linear_kernel,
        grid=(B, P // _CHUNK),
        in_specs=[
            pl.BlockSpec((1, C, _CHUNK), lambda i, j: (i, 0, j)),
            pl.BlockSpec((C, C), lambda i, j: (0, 0)),
            pl.BlockSpec((C, 1), lambda i, j: (0, 0)),
        ],
        out_specs=pl.BlockSpec((1, C, _CHUNK), lambda i, j: (i, 0, j)),
        out_shape=jax.ShapeDtypeStruct((B, C, P), jnp.float32),
    )(x, W, b2)
    return (out.reshape(B, C, H, Wd), feat_map1, feat_map2)
